# Initial kernel scaffold; baseline (speedup 1.0000x reference)
import jax
import jax.numpy as jnp
from jax.experimental import pallas as pl


def kernel(QT, SLI, eta, W_QT, b_QT, W_SLI, b_SLI):
    raise NotImplementedError("write your pallas kernel here")


# speedup vs baseline: 1.6789x; 1.6789x over previous
import jax
import jax.numpy as jnp
from jax.experimental import pallas as pl

_NZ = 34
_ZP = 40
_E = 7
_NB = 2048


def _body(eta_ref, xq_ref, xs_ref, wq_ref, bq_ref, ws_ref, bs_ref, out_ref):
    xq = xq_ref[...]
    xs = xs_ref[...]
    pq = jnp.dot(wq_ref[...], xq, preferred_element_type=jnp.float32)
    ps = jnp.dot(ws_ref[...], xs, preferred_element_type=jnp.float32)
    eta = eta_ref[...]
    nb = xq.shape[1]
    accq = jnp.zeros((_NZ, nb), jnp.float32)
    accs = jnp.zeros((_NZ, nb), jnp.float32)
    for e in range(_E):
        m = (eta == e).astype(jnp.float32)
        accq = accq + (pq[e * _ZP:e * _ZP + _NZ, :] + bq_ref[e * _ZP:e * _ZP + _NZ, :]) * m
        accs = accs + (ps[e * _ZP:e * _ZP + _NZ, :] + bs_ref[e * _ZP:e * _ZP + _NZ, :]) * m
    out_ref[0, :, :] = accq
    out_ref[1, :, :] = accs


def kernel(QT, SLI, eta, W_QT, b_QT, W_SLI, b_SLI):
    nz, ny, nx = QT.shape
    e = W_QT.shape[0]
    n = ny * nx
    xq = QT.reshape(nz, n)
    xs = SLI.reshape(nz, n)
    eta2 = eta.reshape(1, n).astype(jnp.int32)
    pad = ((0, 0), (0, _ZP - nz), (0, 0))
    wq = jnp.pad(W_QT, pad).reshape(e * _ZP, nz)
    ws = jnp.pad(W_SLI, pad).reshape(e * _ZP, nz)
    bpad = ((0, 0), (0, _ZP - nz))
    bq = jnp.pad(b_QT, bpad).reshape(e * _ZP, 1)
    bs = jnp.pad(b_SLI, bpad).reshape(e * _ZP, 1)

    out = pl.pallas_call(
        _body,
        grid=(n // _NB,),
        in_specs=[
            pl.BlockSpec((1, _NB), lambda i: (0, i)),
            pl.BlockSpec((nz, _NB), lambda i: (0, i)),
            pl.BlockSpec((nz, _NB), lambda i: (0, i)),
            pl.BlockSpec((e * _ZP, nz), lambda i: (0, 0)),
            pl.BlockSpec((e * _ZP, 1), lambda i: (0, 0)),
            pl.BlockSpec((e * _ZP, nz), lambda i: (0, 0)),
            pl.BlockSpec((e * _ZP, 1), lambda i: (0, 0)),
        ],
        out_specs=pl.BlockSpec((2, nz, _NB), lambda i: (0, 0, i)),
        out_shape=jax.ShapeDtypeStruct((2, nz, n), jnp.float32),
    )(eta2, xq, xs, wq, bq, ws, bs)
    return out.reshape(2, nz, ny, nx)
